# baseline (speedup 1.0000x reference)
import jax
import jax.numpy as jnp
from jax import lax
from jax.experimental import pallas as pl
from jax.experimental.pallas import tpu as pltpu
from jax.experimental.pallas import tpu_sc as plsc

N_USER = 20000
N_ITEM = 30000
N = N_USER + N_ITEM
NP = 51200
D = 64
HALF = 32
E = 800000
NC = 2
NS = 16
L = 16

SB = 80
SBC = 5
CHUNK = SB * SBC
EDGES_PER_TILE = E // NS
NCHUNK = EDGES_PER_TILE // CHUNK
ACC_ROWS_PER_TILE = NP // NS
FLUSH_STEPS = ACC_ROWS_PER_TILE // CHUNK


def _sc_body(ego, cols, rows, vals, out_b,
             cidx, crow, cval, gath, acc, semI, semG, semS):
    c = lax.axis_index("c")
    s = lax.axis_index("s")
    col_off = c * NP
    ebase = s * EDGES_PER_TILE

    abase0 = s * ACC_ROWS_PER_TILE
    stage = gath.at[1]
    for z in range(FLUSH_STEPS):
        r0 = abase0 + z * CHUNK
        pltpu.sync_copy(ego.at[pl.ds(col_off + r0, CHUNK)], stage)
        pltpu.sync_copy(stage, out_b.at[0, c, pl.ds(r0, CHUNK)])

    z16 = jnp.zeros((L,), jnp.float32)

    def _zero_slab(r, carry):
        gath[0, r, pl.ds(0, L)] = z16
        gath[0, r, pl.ds(L, L)] = z16
        return carry

    lax.fori_loop(0, CHUNK, _zero_slab, None)

    abase = s * ACC_ROWS_PER_TILE
    for z in range(FLUSH_STEPS):
        pltpu.sync_copy(gath.at[0], acc.at[pl.ds(abase + z * CHUNK, CHUNK)])
    plsc.subcore_barrier()

    def fire_index(g, carry_done=None):
        e0 = ebase + g * CHUNK
        sb0 = e0 // SB
        pltpu.async_copy(cols.at[pl.ds(sb0, SBC)], cidx.at[lax.rem(g, 2)], semI)
        pltpu.async_copy(rows.at[pl.ds(sb0, SBC)], crow.at[lax.rem(g, 3)], semI)
        pltpu.async_copy(vals.at[pl.ds(e0, CHUNK)], cval.at[lax.rem(g, 2)], semI)

    def wait_index(g):
        e0 = ebase + g * CHUNK
        sb0 = e0 // SB
        pltpu.make_async_copy(
            cols.at[pl.ds(sb0, SBC)], cidx.at[lax.rem(g, 2)], semI).wait()
        pltpu.make_async_copy(
            rows.at[pl.ds(sb0, SBC)], crow.at[lax.rem(g, 3)], semI).wait()
        pltpu.make_async_copy(
            vals.at[pl.ds(e0, CHUNK)], cval.at[lax.rem(g, 2)], semI).wait()

    def fire_gathers(table, g):
        p = lax.rem(g, 2)
        for j in range(SBC):
            pltpu.async_copy(table.at[cidx.at[p, j]],
                             gath.at[p, pl.ds(j * SB, SB)], semG)

    def drain_gathers(table, g):
        p = lax.rem(g, 2)
        for j in range(SBC):
            pltpu.make_async_copy(table.at[cidx.at[p, j]],
                                  gath.at[p, pl.ds(j * SB, SB)], semG).wait()

    def scale(g):
        p = lax.rem(g, 2)

        def _scale(t, cy):
            valvec = cval[p, pl.ds(t * L, L)]
            for i in range(L):
                r = t * L + i
                v = valvec[i]
                gath[p, r, pl.ds(0, L)] = gath[p, r, pl.ds(0, L)] * v
                gath[p, r, pl.ds(L, L)] = gath[p, r, pl.ds(L, L)] * v
            return cy

        lax.fori_loop(0, CHUNK // L, _scale, None)

    def fire_scatters(g):
        p = lax.rem(g, 2)
        pi = lax.rem(g, 3)
        for j in range(SBC):
            pltpu.async_copy(gath.at[p, pl.ds(j * SB, SB)],
                             acc.at[crow.at[pi, j]], semS, add=True)

    def drain_scatters(g):
        p = lax.rem(g, 2)
        pi = lax.rem(g, 3)
        for j in range(SBC):
            pltpu.make_async_copy(gath.at[p, pl.ds(j * SB, SB)],
                                  acc.at[crow.at[pi, j]], semS).wait()

    def layer(table, kout, rezero):
        fire_index(0)
        wait_index(0)
        fire_gathers(table, 0)
        fire_index(1)

        def chunk_body(g, carry):
            drain_gathers(table, g)

            @pl.when(g + 1 < NCHUNK)
            def _():
                wait_index(g + 1)

                @pl.when(g >= 1)
                def _():
                    drain_scatters(g - 1)

                fire_gathers(table, g + 1)

            scale(g)
            fire_scatters(g)

            @pl.when(g + 2 < NCHUNK)
            def _():
                fire_index(g + 2)
            return carry

        lax.fori_loop(0, NCHUNK, chunk_body, None)
        drain_scatters(NCHUNK - 2)
        drain_scatters(NCHUNK - 1)
        plsc.subcore_barrier()
        if rezero:
            lax.fori_loop(0, CHUNK, _zero_slab, None)
        for z in range(FLUSH_STEPS):
            r0 = abase + z * CHUNK
            pltpu.sync_copy(acc.at[pl.ds(r0, CHUNK)],
                            out_b.at[kout, c, pl.ds(r0, CHUNK)])
            if rezero:
                pltpu.sync_copy(gath.at[0], acc.at[pl.ds(r0, CHUNK)])
        plsc.subcore_barrier()

    layer(out_b.at[0, c], 1, rezero=True)
    layer(out_b.at[1, c], 2, rezero=False)


@jax.jit
def _sc_call(ego_split, cols2, rows2, vals2):
    mesh = plsc.VectorSubcoreMesh(
        core_axis_name="c", subcore_axis_name="s",
        num_cores=NC, num_subcores=NS)
    f = pl.kernel(
        _sc_body,
        out_type=jax.ShapeDtypeStruct((3, NC, NP, HALF), jnp.float32),
        mesh=mesh,
        compiler_params=pltpu.CompilerParams(use_tc_tiling_on_sc=False),
        scratch_types=[
            pltpu.VMEM((2, SBC, SB), jnp.int32),
            pltpu.VMEM((3, SBC, SB), jnp.int32),
            pltpu.VMEM((2, CHUNK), jnp.float32),
            pltpu.VMEM((2, CHUNK, HALF), jnp.float32),
            pltpu.VMEM_SHARED((NP, HALF), jnp.float32),
            pltpu.SemaphoreType.DMA,
            pltpu.SemaphoreType.DMA,
            pltpu.SemaphoreType.DMA,
        ],
    )
    return f(ego_split, cols2, rows2, vals2)


def kernel(user_emb, item_emb, hg_rows, hg_cols, hg_vals):
    ego = jnp.concatenate([user_emb, item_emb], axis=0)
    pad = jnp.zeros((NP - N, HALF), jnp.float32)
    ego_split = jnp.concatenate(
        [ego[:, :HALF], pad, ego[:, HALF:], pad], axis=0)
    cols2 = hg_cols.reshape(E // SB, SB)
    rows2 = hg_rows.reshape(E // SB, SB)
    vals2 = hg_vals
    out_b = _sc_call(ego_split, cols2, rows2, vals2)
    e = out_b[:, :, :N, :]
    all_emb = e.transpose(2, 0, 1, 3).reshape(N, 3 * D)
    return all_emb[:N_USER], all_emb[N_USER:]
